# Initial kernel scaffold; baseline (speedup 1.0000x reference)
import jax
import jax.numpy as jnp
from jax import lax
from jax.experimental import pallas as pl
from jax.experimental.pallas import tpu as pltpu
from jax.experimental.pallas import tpu_sc as plsc

B = 16384
D = 16
L = 50
NE = 8
F = (NE + 1) * D
NC = 2
NS = 16
NW = NC * NS
S = B // NW
C = 128
NCH = S // C

HIST_COL = NE * D


def _sc_gather_body(idx8_hbm, hist_hbm,
                    emb_user, emb_item, ec0, ec1, ec2, ec3, ec4, ec5,
                    emb_hist,
                    feats_hbm,
                    idx8_v, hidx_v, rows8_v, hrows_v, feats_v,
                    sem_g, sem_h):
    tables = (emb_user, emb_item, ec0, ec1, ec2, ec3, ec4, ec5)
    wid = lax.axis_index("s") * NC + lax.axis_index("c")

    @pl.loop(0, NCH)
    def _chunk(c):
        base = wid * S + c * C

        pltpu.sync_copy(idx8_hbm.at[:, pl.ds(base, C)], idx8_v)
        pltpu.sync_copy(hist_hbm.at[:, pl.ds(base, C)], hidx_v)

        descs = []
        for t in range(NE):
            descs.append(pltpu.async_copy(
                tables[t].at[idx8_v.at[t]], rows8_v.at[t], sem_g))

        @pl.loop(0, C, step=16)
        def _zero(r0):
            for dr in range(16):
                feats_v[r0 + dr, pl.ds(HIST_COL, D)] = jnp.zeros(
                    (D,), jnp.float32)

        @pl.loop(0, L)
        def _hist(k):
            pltpu.async_copy(
                emb_hist.at[hidx_v.at[k]], hrows_v, sem_h).wait()

            @pl.loop(0, C, step=16)
            def _acc(r0):
                for dr in range(16):
                    r = r0 + dr
                    plsc.addupdate(
                        feats_v.at[r, pl.ds(HIST_COL, D)], hrows_v[r, :])

        for t in range(NE):
            descs[t].wait()
        for t in range(NE):
            @pl.loop(0, C, step=16)
            def _place(r0, t=t):
                for dr in range(16):
                    r = r0 + dr
                    feats_v[r, pl.ds(t * D, D)] = rows8_v[t, r, :]

        pltpu.sync_copy(feats_v, feats_hbm.at[pl.ds(base, C), :])


def _sc_gather(idx8, histT, emb_user, emb_item, ec0, ec1, ec2, ec3, ec4,
               ec5, emb_hist):
    mesh = plsc.VectorSubcoreMesh(core_axis_name="c", subcore_axis_name="s")
    return pl.kernel(
        _sc_gather_body,
        out_type=jax.ShapeDtypeStruct((B, F), jnp.float32),
        mesh=mesh,
        scratch_types=[
            pltpu.VMEM((NE, C), jnp.int32),
            pltpu.VMEM((L, C), jnp.int32),
            pltpu.VMEM((NE, C, D), jnp.float32),
            pltpu.VMEM((C, D), jnp.float32),
            pltpu.VMEM((C, F), jnp.float32),
            pltpu.SemaphoreType.DMA,
            pltpu.SemaphoreType.DMA,
        ],
    )(idx8, histT, emb_user, emb_item, ec0, ec1, ec2, ec3, ec4, ec5,
      emb_hist)


def _mlp_body(x_ref, ctn_ref, wv_ref, W1_ref, b1_ref, W2_ref, b2_ref,
              W3_ref, b3_ref, o_ref):
    x = x_ref[...]
    h = jnp.maximum(x @ W1_ref[...] + b1_ref[...][None, :], 0.0)
    h = jnp.maximum(h @ W2_ref[...] + b2_ref[...][None, :], 0.0)
    z = h @ W3_ref[...]
    lin = ctn_ref[...] @ wv_ref[...]
    r = z[:, 0] + lin[:, 0] + b3_ref[0]
    o_ref[...] = jax.nn.sigmoid(r)


def _mlp(feats, ctn, wvec, W1, b1, W2, b2, W3, b3):
    bm = 2048
    grid = (B // bm,)
    return pl.pallas_call(
        _mlp_body,
        grid=grid,
        in_specs=[
            pl.BlockSpec((bm, F), lambda i: (i, 0)),
            pl.BlockSpec((bm, 4), lambda i: (i, 0)),
            pl.BlockSpec((4, 1), lambda i: (0, 0)),
            pl.BlockSpec((F, 256), lambda i: (0, 0)),
            pl.BlockSpec((256,), lambda i: (0,)),
            pl.BlockSpec((256, 128), lambda i: (0, 0)),
            pl.BlockSpec((128,), lambda i: (0,)),
            pl.BlockSpec((128, 1), lambda i: (0, 0)),
            pl.BlockSpec((1,), lambda i: (0,)),
        ],
        out_specs=pl.BlockSpec((bm,), lambda i: (i,)),
        out_shape=jax.ShapeDtypeStruct((B,), jnp.float32),
    )(feats, ctn, wvec, W1, b1, W2, b2, W3, b3)


def kernel(user_id, item_id, cat_0, cat_1, cat_2, cat_3, cat_4, cat_5,
           ctn_0, ctn_1, ctn_2, ctn_3, hist_item,
           emb_user, emb_item, emb_cat_0, emb_cat_1, emb_cat_2, emb_cat_3,
           emb_cat_4, emb_cat_5, emb_hist,
           w_ctn_0, w_ctn_1, w_ctn_2, w_ctn_3,
           W1, b1, W2, b2, W3, b3):
    idx8 = jnp.stack([
        user_id[:, 0], item_id[:, 0], cat_0[:, 0], cat_1[:, 0],
        cat_2[:, 0], cat_3[:, 0], cat_4[:, 0], cat_5[:, 0],
    ]).astype(jnp.int32)
    histT = hist_item.T.astype(jnp.int32)

    feats = _sc_gather(idx8, histT, emb_user, emb_item, emb_cat_0,
                       emb_cat_1, emb_cat_2, emb_cat_3, emb_cat_4,
                       emb_cat_5, emb_hist)

    ctn = jnp.concatenate([ctn_0, ctn_1, ctn_2, ctn_3], axis=1)
    wvec = jnp.stack([w_ctn_0[0, 0], w_ctn_1[0, 0], w_ctn_2[0, 0],
                      w_ctn_3[0, 0]]).reshape(4, 1)
    return _mlp(feats, ctn, wvec, W1, b1, W2, b2, W3, b3)


# speedup vs baseline: 1.2301x; 1.2301x over previous
import jax
import jax.numpy as jnp
from jax import lax
from jax.experimental import pallas as pl
from jax.experimental.pallas import tpu as pltpu
from jax.experimental.pallas import tpu_sc as plsc

B = 16384
D = 16
L = 50
NE = 8
F = (NE + 1) * D
NC = 2
NS = 16
NW = NC * NS
S = B // NW
C = 128
NCH = S // C

HIST_COL = NE * D


def _sc_gather_body(idx8_hbm, hist_hbm,
                    emb_user, emb_item, ec0, ec1, ec2, ec3, ec4, ec5,
                    emb_hist,
                    feats_hbm,
                    idx8_v, hidx_v, rows8_v, hrows_v, feats_v,
                    sem_g, sem_h):
    tables = (emb_user, emb_item, ec0, ec1, ec2, ec3, ec4, ec5)
    wid = lax.axis_index("s") * NC + lax.axis_index("c")

    @pl.loop(0, NCH)
    def _chunk(c):
        base = wid * S + c * C

        pltpu.sync_copy(idx8_hbm.at[:, pl.ds(base, C)], idx8_v)
        pltpu.sync_copy(hist_hbm.at[:, pl.ds(base, C)], hidx_v)

        descs = []
        for t in range(NE):
            descs.append(pltpu.async_copy(
                tables[t].at[idx8_v.at[t]], rows8_v.at[t], sem_g))

        @pl.loop(0, C, step=16)
        def _zero(r0):
            for dr in range(16):
                feats_v[r0 + dr, pl.ds(HIST_COL, D)] = jnp.zeros(
                    (D,), jnp.float32)

        @pl.loop(0, L)
        def _hist(k):
            pltpu.async_copy(
                emb_hist.at[hidx_v.at[k]], hrows_v, sem_h).wait()

            @pl.loop(0, C, step=16)
            def _acc(r0):
                for dr in range(16):
                    r = r0 + dr
                    plsc.addupdate(
                        feats_v.at[r, pl.ds(HIST_COL, D)], hrows_v[r, :])

        for t in range(NE):
            descs[t].wait()
        for t in range(NE):
            @pl.loop(0, C, step=16)
            def _place(r0, t=t):
                for dr in range(16):
                    r = r0 + dr
                    feats_v[r, pl.ds(t * D, D)] = rows8_v[t, r, :]

        pltpu.sync_copy(feats_v, feats_hbm.at[pl.ds(base, C), :])


def _sc_gather(idx8, histT, emb_user, emb_item, ec0, ec1, ec2, ec3, ec4,
               ec5, emb_hist):
    mesh = plsc.VectorSubcoreMesh(core_axis_name="c", subcore_axis_name="s")
    return pl.kernel(
        _sc_gather_body,
        out_type=jax.ShapeDtypeStruct((B, F), jnp.float32),
        mesh=mesh,
        scratch_types=[
            pltpu.VMEM((NE, C), jnp.int32),
            pltpu.VMEM((L, C), jnp.int32),
            pltpu.VMEM((NE, C, D), jnp.float32),
            pltpu.VMEM((C, D), jnp.float32),
            pltpu.VMEM((C, F), jnp.float32),
            pltpu.SemaphoreType.DMA,
            pltpu.SemaphoreType.DMA,
        ],
        compiler_params=pltpu.CompilerParams(use_tc_tiling_on_sc=False),
    )(idx8, histT, emb_user, emb_item, ec0, ec1, ec2, ec3, ec4, ec5,
      emb_hist)


def _mlp_body(x_ref, ctn_ref, wv_ref, W1_ref, b1_ref, W2_ref, b2_ref,
              W3_ref, b3_ref, o_ref):
    x = x_ref[...]
    h = jnp.maximum(x @ W1_ref[...] + b1_ref[...][None, :], 0.0)
    h = jnp.maximum(h @ W2_ref[...] + b2_ref[...][None, :], 0.0)
    z = h @ W3_ref[...]
    lin = ctn_ref[...] @ wv_ref[...]
    r = z[:, 0] + lin[:, 0] + b3_ref[0]
    o_ref[...] = jax.nn.sigmoid(r)


def _mlp(feats, ctn, wvec, W1, b1, W2, b2, W3, b3):
    bm = 2048
    grid = (B // bm,)
    return pl.pallas_call(
        _mlp_body,
        grid=grid,
        in_specs=[
            pl.BlockSpec((bm, F), lambda i: (i, 0)),
            pl.BlockSpec((bm, 4), lambda i: (i, 0)),
            pl.BlockSpec((4, 1), lambda i: (0, 0)),
            pl.BlockSpec((F, 256), lambda i: (0, 0)),
            pl.BlockSpec((256,), lambda i: (0,)),
            pl.BlockSpec((256, 128), lambda i: (0, 0)),
            pl.BlockSpec((128,), lambda i: (0,)),
            pl.BlockSpec((128, 1), lambda i: (0, 0)),
            pl.BlockSpec((1,), lambda i: (0,)),
        ],
        out_specs=pl.BlockSpec((bm,), lambda i: (i,)),
        out_shape=jax.ShapeDtypeStruct((B,), jnp.float32),
    )(feats, ctn, wvec, W1, b1, W2, b2, W3, b3)


def kernel(user_id, item_id, cat_0, cat_1, cat_2, cat_3, cat_4, cat_5,
           ctn_0, ctn_1, ctn_2, ctn_3, hist_item,
           emb_user, emb_item, emb_cat_0, emb_cat_1, emb_cat_2, emb_cat_3,
           emb_cat_4, emb_cat_5, emb_hist,
           w_ctn_0, w_ctn_1, w_ctn_2, w_ctn_3,
           W1, b1, W2, b2, W3, b3):
    idx8 = jnp.stack([
        user_id[:, 0], item_id[:, 0], cat_0[:, 0], cat_1[:, 0],
        cat_2[:, 0], cat_3[:, 0], cat_4[:, 0], cat_5[:, 0],
    ]).astype(jnp.int32)
    histT = hist_item.T.astype(jnp.int32)

    feats = _sc_gather(idx8, histT, emb_user, emb_item, emb_cat_0,
                       emb_cat_1, emb_cat_2, emb_cat_3, emb_cat_4,
                       emb_cat_5, emb_hist)

    ctn = jnp.concatenate([ctn_0, ctn_1, ctn_2, ctn_3], axis=1)
    wvec = jnp.stack([w_ctn_0[0, 0], w_ctn_1[0, 0], w_ctn_2[0, 0],
                      w_ctn_3[0, 0]]).reshape(4, 1)
    return _mlp(feats, ctn, wvec, W1, b1, W2, b2, W3, b3)
